# baseline (speedup 1.0000x reference)
import jax
import jax.numpy as jnp
from jax.experimental import pallas as pl
from jax.experimental.pallas import tpu as pltpu

V = 4096
D = 128
B = 8
S = 32
P = 8
CONC = 5.0


def _kernel(idx_ref, clen_ref, seg_ref,
            embed_ref, kb_ref, aff_ref, lam_ref,
            e0, e1, e2, e3, e4, e5, e6, e7,
            out_ref, a_scr, c_scr):
    b = pl.program_id(0)
    s = pl.program_id(1)

    @pl.when(jnp.logical_and(b == 0, s == 0))
    def _prologue():
        cn_rows = []
        for bb in range(B):
            segb = seg_ref[bb]

            def body(i, acc):
                ss = i // P
                pp = i - ss * P
                L = clen_ref[bb * S + ss]
                c = idx_ref[(bb * S + ss) * P + pp]
                valid = jnp.logical_and(ss < segb, pp < L)
                wgt = jnp.where(valid, 1.0, 0.0) / (
                    jnp.maximum(L, 1) * segb).astype(jnp.float32)
                return acc + wgt * embed_ref[pl.ds(c, 1), :]

            ctx = jax.lax.fori_loop(0, S * P, body,
                                    jnp.zeros((1, D), jnp.float32))
            nrm = jnp.sqrt(jnp.sum(ctx * ctx))
            cn_rows.append(ctx / jnp.maximum(nrm, 1e-8))
        cnm = jnp.concatenate(cn_rows, axis=0)
        kb = kb_ref[:, :]
        kn2 = jnp.sum(kb * kb, axis=1, keepdims=True)
        kbn = kb / jnp.maximum(jnp.sqrt(kn2), 1e-8)
        dots = jax.lax.dot_general(
            cnm, kbn, (((1,), (1,)), ((), ())),
            preferred_element_type=jnp.float32)
        lamv = lam_ref[:, :]
        a_scr[:, :] = jnp.abs(dots) * (CONC * lamv)
        c_scr[:, :] = CONC * (1.0 - lamv) * aff_ref[:, :]

    rows = jnp.concatenate(
        [e0[0], e1[0], e2[0], e3[0],
         e4[0], e5[0], e6[0], e7[0]], axis=0)
    av = a_scr[pl.ds(b, 1), :]
    cv = c_scr[:, :]
    w = rows * av + jnp.where(rows > 0, cv, 0.0)
    m = jnp.max(w, axis=1, keepdims=True)
    e = jnp.exp(w - m)
    ssum = jnp.sum(e, axis=1, keepdims=True)
    y = jnp.dot(e, kb_ref[:, :],
                preferred_element_type=jnp.float32)
    L = clen_ref[b * S + s]
    pm = jax.lax.broadcasted_iota(jnp.int32, (P, D), 0) < L
    y = jnp.where(pm, y / ssum, 0.0)
    outrow = jnp.sum(y, axis=0, keepdims=True) / jnp.maximum(
        L, 1).astype(jnp.float32)
    segb = seg_ref[b]
    out_ref[0] = jnp.where(s < segb, outrow, 0.0)


@jax.jit
def kernel(concepts, concepts_length, seg_len, embed, embed_kb,
           edge_matrix, affectiveness, lam):
    idx = concepts.reshape(-1).astype(jnp.int32)
    clen = concepts_length.reshape(-1).astype(jnp.int32)
    seg = seg_len.astype(jnp.int32)
    aff2 = affectiveness.reshape(1, V)
    lam2 = lam.reshape(1, V)

    full = lambda b, s, *_: (0, 0)

    def edge_map(p):
        return lambda b, s, idx_ref, clen_ref, seg_ref: (
            idx_ref[(b * S + s) * P + p], 0, 0)

    edge3 = edge_matrix.reshape(V, 1, V)

    out = pl.pallas_call(
        _kernel,
        grid_spec=pltpu.PrefetchScalarGridSpec(
            num_scalar_prefetch=3,
            grid=(B, S),
            in_specs=[
                pl.BlockSpec((V, D), full),
                pl.BlockSpec((V, D), full),
                pl.BlockSpec((1, V), full),
                pl.BlockSpec((1, V), full),
            ] + [pl.BlockSpec((1, 1, V), edge_map(p)) for p in range(P)],
            out_specs=pl.BlockSpec((1, 1, D),
                                   lambda b, s, *_: (b * S + s, 0, 0)),
            scratch_shapes=[
                pltpu.VMEM((B, V), jnp.float32),
                pltpu.VMEM((1, V), jnp.float32),
            ],
        ),
        out_shape=jax.ShapeDtypeStruct((B * S, 1, D), jnp.float32),
    )(idx, clen, seg, embed, embed_kb, aff2, lam2,
      *([edge3] * P))
    return out.reshape(B, S, D)


# speedup vs baseline: 1.3894x; 1.3894x over previous
import jax
import jax.numpy as jnp
from jax.experimental import pallas as pl
from jax.experimental.pallas import tpu as pltpu

V = 4096
D = 128
B = 8
S = 32
P = 8
CONC = 5.0


def _kernel(idx_ref, clen_ref, seg_ref,
            embed_ref, kb_ref, aff_ref, lam_ref,
            e0, e1, e2, e3, e4, e5, e6, e7,
            out_ref, a_scr, c_scr, g_scr):
    b = pl.program_id(0)
    s = pl.program_id(1)
    step = b * S + s

    @pl.when(step == 0)
    def _prologue():
        cn_rows = []
        for bb in range(B):
            segb = seg_ref[bb]

            def body(i, acc):
                ss = i // P
                pp = i - ss * P
                L = clen_ref[bb * S + ss]
                c = idx_ref[(bb * S + ss) * P + pp]
                wgt = jnp.where(pp < L, 1.0, 0.0) / (
                    jnp.maximum(L, 1) * segb).astype(jnp.float32)
                return acc + wgt * embed_ref[pl.ds(c, 1), :]

            ctx = jax.lax.fori_loop(0, segb * P, body,
                                    jnp.zeros((1, D), jnp.float32))
            nrm = jnp.sqrt(jnp.sum(ctx * ctx))
            cn_rows.append(ctx / jnp.maximum(nrm, 1e-8))
        cnm = jnp.concatenate(cn_rows, axis=0)
        kb = kb_ref[:, :]
        kn2 = jnp.sum(kb * kb, axis=1, keepdims=True)
        kbn = kb / jnp.maximum(jnp.sqrt(kn2), 1e-8)
        dots = jax.lax.dot_general(
            cnm, kbn, (((1,), (1,)), ((), ())),
            preferred_element_type=jnp.float32)
        lamv = lam_ref[:, :]
        a_scr[:, :] = jnp.abs(dots) * (CONC * lamv)
        c_scr[:, :] = CONC * (1.0 - lamv) * aff_ref[:, :]

    segb = seg_ref[b]

    @pl.when(s < segb)
    def _pos():
        rows = jnp.concatenate(
            [e0[0], e1[0], e2[0], e3[0],
             e4[0], e5[0], e6[0], e7[0]], axis=0)
        av = a_scr[pl.ds(b, 1), :]
        cv = c_scr[:, :]
        e = jnp.exp(rows * av + jnp.where(rows > 0, cv, 0.0))
        ssum = jnp.sum(e, axis=1, keepdims=True)
        L = clen_ref[step]
        pm = jax.lax.broadcasted_iota(jnp.int32, (P, 1), 0) < L
        scale = jnp.where(pm, 1.0, 0.0) / (
            ssum * jnp.maximum(L, 1).astype(jnp.float32))
        g_scr[pl.ds(step, 1), :] = jnp.sum(e * scale, axis=0,
                                           keepdims=True)

    @pl.when(s >= segb)
    def _pad():
        g_scr[pl.ds(step, 1), :] = jnp.zeros((1, V), jnp.float32)

    @pl.when(step == B * S - 1)
    def _epilogue():
        out_ref[:, :] = jnp.dot(g_scr[:, :], kb_ref[:, :],
                                preferred_element_type=jnp.float32)


@jax.jit
def kernel(concepts, concepts_length, seg_len, embed, embed_kb,
           edge_matrix, affectiveness, lam):
    conc = concepts.astype(jnp.int32)
    clen2 = concepts_length.astype(jnp.int32)
    seg = seg_len.astype(jnp.int32)

    valid = jnp.logical_and(
        jnp.arange(P)[None, None, :] < clen2[:, :, None],
        jnp.arange(S)[None, :, None] < seg[:, None, None])

    def ffill(carry, x):
        c, v = x
        newc = jnp.where(v, c, carry)
        return newc, newc

    _, c2 = jax.lax.scan(ffill, conc[:, 0],
                         (conc.swapaxes(0, 1), valid.swapaxes(0, 1)))
    idx = c2.swapaxes(0, 1).reshape(-1)

    clen = clen2.reshape(-1)
    aff2 = affectiveness.reshape(1, V)
    lam2 = lam.reshape(1, V)
    edge3 = edge_matrix.reshape(V, 1, V)

    full = lambda b, s, *_: (0, 0)

    def edge_map(p):
        return lambda b, s, idx_ref, clen_ref, seg_ref: (
            idx_ref[(b * S + s) * P + p], 0, 0)

    out = pl.pallas_call(
        _kernel,
        grid_spec=pltpu.PrefetchScalarGridSpec(
            num_scalar_prefetch=3,
            grid=(B, S),
            in_specs=[
                pl.BlockSpec((V, D), full),
                pl.BlockSpec((V, D), full),
                pl.BlockSpec((1, V), full),
                pl.BlockSpec((1, V), full),
            ] + [pl.BlockSpec((1, 1, V), edge_map(p)) for p in range(P)],
            out_specs=pl.BlockSpec((B * S, D), lambda b, s, *_: (0, 0)),
            scratch_shapes=[
                pltpu.VMEM((B, V), jnp.float32),
                pltpu.VMEM((1, V), jnp.float32),
                pltpu.VMEM((B * S, V), jnp.float32),
            ],
        ),
        out_shape=jax.ShapeDtypeStruct((B * S, D), jnp.float32),
    )(idx, clen, seg, embed, embed_kb, aff2, lam2,
      *([edge3] * P))
    return out.reshape(B, S, D)


# speedup vs baseline: 1.4566x; 1.0484x over previous
import jax
import jax.numpy as jnp
from jax.experimental import pallas as pl
from jax.experimental.pallas import tpu as pltpu

V = 4096
D = 128
B = 8
S = 32
P = 8
CONC = 5.0
G = 4
Q = 8
W = V // Q


def _gather_softmax_kernel(idx_ref, clen_ref, seg_ref,
                           embed_ref, kb_ref, aff_ref, lam_ref,
                           *rest):
    erefs = rest[:G * P]
    out_ref = rest[G * P]
    a_scr, c_scr = rest[G * P + 1:]
    b = pl.program_id(0)
    g = pl.program_id(1)

    @pl.when(jnp.logical_and(b == 0, g == 0))
    def _prologue():
        cn_rows = []
        for bb in range(B):
            segb = seg_ref[bb]

            def body(i, acc):
                ss = i // P
                pp = i - ss * P
                L = clen_ref[bb * S + ss]
                c = idx_ref[(bb * S + ss) * P + pp]
                wgt = jnp.where(pp < L, 1.0, 0.0) / (
                    jnp.maximum(L, 1) * segb).astype(jnp.float32)
                return acc + wgt * embed_ref[pl.ds(c, 1), :]

            ctx = jax.lax.fori_loop(0, segb * P, body,
                                    jnp.zeros((1, D), jnp.float32))
            nrm = jnp.sqrt(jnp.sum(ctx * ctx))
            cn_rows.append(ctx / jnp.maximum(nrm, 1e-8))
        cnm = jnp.concatenate(cn_rows, axis=0)
        kb = kb_ref[:, :]
        kn2 = jnp.sum(kb * kb, axis=1, keepdims=True)
        kbn = kb / jnp.maximum(jnp.sqrt(kn2), 1e-8)
        dots = jax.lax.dot_general(
            cnm, kbn, (((1,), (1,)), ((), ())),
            preferred_element_type=jnp.float32)
        lamv = lam_ref[:, :]
        a_scr[:, :] = jnp.abs(dots) * (CONC * lamv)
        cvec = CONC * (1.0 - lamv) * aff_ref[:, :]
        c_scr[:, :] = jnp.concatenate(
            [cvec[:, q * W:(q + 1) * W] for q in range(Q)], axis=0)

    segb = seg_ref[b]

    @pl.when(g * G < segb)
    def _work():
        av = jnp.concatenate(
            [a_scr[pl.ds(b, 1), pl.ds(q * W, W)] for q in range(Q)],
            axis=0)
        cv = c_scr[:, :]
        accs = []
        for t in range(G):
            spos = g * G + t
            pos = b * S + spos
            L = clen_ref[pos]
            within = spos < segb
            inv_den = jnp.where(
                within, 1.0, 0.0) / jnp.maximum(L, 1).astype(jnp.float32)
            acc = jnp.zeros((Q, W), jnp.float32)
            for p in range(P):
                row = erefs[t * P + p][0]
                e = jnp.exp(row * av + jnp.where(row > 0, cv, 0.0))
                sm = jnp.sum(e)
                scale = jnp.where(p < L, inv_den, 0.0) / sm
                acc = acc + scale * e
            accs.append(acc)
        out_ref[:, :] = jnp.concatenate(accs, axis=0)

    @pl.when(g * G >= segb)
    def _pad():
        out_ref[:, :] = jnp.zeros((G * Q, W), jnp.float32)


def _matmul_kernel(g_ref, kb_ref, out_ref):
    out_ref[:, :] = jnp.dot(g_ref[:, :], kb_ref[:, :],
                            preferred_element_type=jnp.float32)


@jax.jit
def kernel(concepts, concepts_length, seg_len, embed, embed_kb,
           edge_matrix, affectiveness, lam):
    conc = concepts.astype(jnp.int32)
    clen2 = concepts_length.astype(jnp.int32)
    seg = seg_len.astype(jnp.int32)

    valid = jnp.logical_and(
        jnp.arange(P)[None, None, :] < clen2[:, :, None],
        jnp.arange(S)[None, :, None] < seg[:, None, None])

    def ffill(carry, x):
        c, v = x
        newc = jnp.where(v, c, carry)
        return newc, newc

    _, c2 = jax.lax.scan(ffill, conc[:, 0],
                         (conc.swapaxes(0, 1), valid.swapaxes(0, 1)))
    idx = c2.swapaxes(0, 1).reshape(-1)

    clen = clen2.reshape(-1)
    aff2 = affectiveness.reshape(1, V)
    lam2 = lam.reshape(1, V)
    edge4 = edge_matrix.reshape(V, Q, W)

    full = lambda b, g, *_: (0, 0)

    def edge_map(t, p):
        return lambda b, g, idx_ref, clen_ref, seg_ref: (
            idx_ref[(b * S + g * G + t) * P + p], 0, 0)

    gmat = pl.pallas_call(
        _gather_softmax_kernel,
        grid_spec=pltpu.PrefetchScalarGridSpec(
            num_scalar_prefetch=3,
            grid=(B, S // G),
            in_specs=[
                pl.BlockSpec((V, D), full),
                pl.BlockSpec((V, D), full),
                pl.BlockSpec((1, V), full),
                pl.BlockSpec((1, V), full),
            ] + [pl.BlockSpec((1, Q, W), edge_map(t, p))
                 for t in range(G) for p in range(P)],
            out_specs=pl.BlockSpec((G * Q, W),
                                   lambda b, g, *_: (b * (S // G) + g, 0)),
            scratch_shapes=[
                pltpu.VMEM((B, V), jnp.float32),
                pltpu.VMEM((Q, W), jnp.float32),
            ],
        ),
        out_shape=jax.ShapeDtypeStruct((B * S * Q, W), jnp.float32),
    )(idx, clen, seg, embed, embed_kb, aff2, lam2,
      *([edge4] * (G * P)))

    gmat2 = gmat.reshape(B * S, V)
    out = pl.pallas_call(
        _matmul_kernel,
        grid=(1,),
        in_specs=[
            pl.BlockSpec((B * S, V), lambda i: (0, 0)),
            pl.BlockSpec((V, D), lambda i: (0, 0)),
        ],
        out_specs=pl.BlockSpec((B * S, D), lambda i: (0, 0)),
        out_shape=jax.ShapeDtypeStruct((B * S, D), jnp.float32),
    )(gmat2, embed_kb)
    return out.reshape(B, S, D)


# speedup vs baseline: 7.1321x; 4.8964x over previous
import jax
import jax.numpy as jnp
from jax.experimental import pallas as pl
from jax.experimental.pallas import tpu as pltpu

V = 4096
D = 128
B = 8
S = 32
P = 8
CONC = 5.0
T = 8
NG = B * S // T
NB = 4


def _kernel(idx_ref, seg_ref, pmds_ref,
            pmdv_ref, embed_ref, kb_ref, aff_ref, lam_ref, edge_ref,
            out_ref, buf, a_scr, c_scr, g_scr, sem):
    i = pl.program_id(0)

    def issue(gi, slot):
        for t in range(T):
            for p in range(P):
                c = idx_ref[(gi * T + t) * P + p]
                pltpu.make_async_copy(
                    edge_ref.at[pl.ds(c, 1), :],
                    buf.at[slot, p, pl.ds(t, 1), :],
                    sem.at[slot, p, t]).start()

    @pl.when(i == 0)
    def _prologue():
        for k in range(NB - 1):
            issue(k, k)
        cn_rows = []
        for bb in range(B):
            segb = seg_ref[bb]

            def body(j, acc):
                c = idx_ref[bb * S * P + j]
                wgt = pmds_ref[bb * S * P + j]
                return acc + wgt * embed_ref[pl.ds(c, 1), :]

            ctx = jax.lax.fori_loop(0, segb * P, body,
                                    jnp.zeros((1, D), jnp.float32))
            ctx = ctx / segb.astype(jnp.float32)
            nrm = jnp.sqrt(jnp.sum(ctx * ctx))
            cn_rows.append(ctx / jnp.maximum(nrm, 1e-8))
        cnm = jnp.concatenate(cn_rows, axis=0)
        kb = kb_ref[:, :]
        kn2 = jnp.sum(kb * kb, axis=1, keepdims=True)
        kbn = kb / jnp.maximum(jnp.sqrt(kn2), 1e-8)
        dots = jax.lax.dot_general(
            cnm, kbn, (((1,), (1,)), ((), ())),
            preferred_element_type=jnp.float32)
        lamv = lam_ref[:, :]
        a_scr[:, :] = jnp.abs(dots) * (CONC * lamv)
        c_scr[:, :] = CONC * (1.0 - lamv) * aff_ref[:, :]

    gi2 = i + NB - 1

    @pl.when(gi2 < NG)
    def _issue_ahead():
        issue(gi2, jax.lax.rem(gi2, NB))

    slot = jax.lax.rem(i, NB)
    for t in range(T):
        for p in range(P):
            c = idx_ref[(i * T + t) * P + p]
            pltpu.make_async_copy(
                edge_ref.at[pl.ds(c, 1), :],
                buf.at[slot, p, pl.ds(t, 1), :],
                sem.at[slot, p, t]).wait()

    b = i // (S // T)
    av = a_scr[pl.ds(b, 1), :]
    cv = c_scr[:, :]
    pmd = pmdv_ref[0]
    acc = jnp.zeros((T, V), jnp.float32)
    for p in range(P):
        rows = buf[slot, p]
        e = jnp.exp(rows * av + jnp.where(rows > 0, cv, 0.0))
        ssum = jnp.sum(e, axis=1, keepdims=True)
        colp = pmd[:, p:p + 1]
        acc = acc + jnp.where(colp > 0, e * (colp / ssum), 0.0)
    g_scr[pl.ds(i * T, T), :] = acc

    @pl.when(i == NG - 1)
    def _epilogue():
        out_ref[:, :] = jnp.dot(g_scr[:, :], kb_ref[:, :],
                                preferred_element_type=jnp.float32)


@jax.jit
def kernel(concepts, concepts_length, seg_len, embed, embed_kb,
           edge_matrix, affectiveness, lam):
    idx = concepts.astype(jnp.int32).reshape(-1)
    clen2 = concepts_length.astype(jnp.int32)
    seg = seg_len.astype(jnp.int32)

    pmask = jnp.logical_and(
        jnp.arange(P)[None, None, :] < clen2[:, :, None],
        jnp.arange(S)[None, :, None] < seg[:, None, None]).astype(
            jnp.float32)
    pmd = (pmask / jnp.maximum(clen2, 1)[:, :, None]).reshape(-1)
    pmdv = pmd.reshape(NG, T, P)

    aff2 = affectiveness.reshape(1, V)
    lam2 = lam.reshape(1, V)

    full = lambda i, *_: (0, 0)

    out = pl.pallas_call(
        _kernel,
        grid_spec=pltpu.PrefetchScalarGridSpec(
            num_scalar_prefetch=3,
            grid=(NG,),
            in_specs=[
                pl.BlockSpec((1, T, P), lambda i, *_: (i, 0, 0)),
                pl.BlockSpec((V, D), full),
                pl.BlockSpec((V, D), full),
                pl.BlockSpec((1, V), full),
                pl.BlockSpec((1, V), full),
                pl.BlockSpec(memory_space=pl.ANY),
            ],
            out_specs=pl.BlockSpec((B * S, D), lambda i, *_: (0, 0)),
            scratch_shapes=[
                pltpu.VMEM((NB, P, T, V), jnp.float32),
                pltpu.VMEM((B, V), jnp.float32),
                pltpu.VMEM((1, V), jnp.float32),
                pltpu.VMEM((B * S, V), jnp.float32),
                pltpu.SemaphoreType.DMA((NB, P, T)),
            ],
        ),
        out_shape=jax.ShapeDtypeStruct((B * S, D), jnp.float32),
    )(idx, seg, pmd, pmdv, embed, embed_kb, aff2, lam2, edge_matrix)
    return out.reshape(B, S, D)
